# baseline (speedup 1.0000x reference)
import numpy as np
import jax
import jax.numpy as jnp
from jax.experimental import pallas as pl

_NMOL = 64
_MAXA = 32
_NSP = 4
_NRS2 = 24
_NRS3 = 20
_RCUT = 8.0
_ETA2 = 0.32
_ETA3 = 2.7
_TWO_BODY_DECAY = 1.8
_THREE_BODY_DECAY = 0.57
_W3 = float(np.sqrt(_ETA3 / np.pi) * 13.4)
_RS2 = np.linspace(0.0, _RCUT, _NRS2 + 1)[1:].astype(np.float32)
_RS3 = np.linspace(0.0, _RCUT, _NRS3 + 1)[1:].astype(np.float32)
_SPECIES = np.array([1.0, 6.0, 7.0, 8.0], dtype=np.float32)
_PAIRS = [(0, 0), (0, 1), (0, 2), (0, 3), (1, 1), (1, 2), (1, 3),
          (2, 2), (2, 3), (3, 3)]
_NPAIR = len(_PAIRS)
_FP = _NSP * _NRS2 + _NPAIR * _NRS3 * 2


def _mol_kernel(x_ref, z_ref, out_ref):
    A = _MAXA
    x = x_ref[0]
    z = z_ref[0, 0]

    diff = x[:, None, :] - x[None, :, :]
    d2 = jnp.sum(diff * diff, axis=-1)
    eye = jnp.eye(A, dtype=jnp.bool_)
    r = jnp.sqrt(jnp.where(eye, 1.0, d2) + 1e-12)
    ir = 1.0 / r
    in_cut = (~eye) & (r < _RCUT)
    fc = jnp.where(in_cut, 0.5 * (jnp.cos(jnp.pi * (1.0 / _RCUT) * r) + 1.0), 0.0)
    lnr = jnp.log(r)
    pd = jnp.exp(-_THREE_BODY_DECAY * lnr)

    oh = jnp.stack([(z == float(s)).astype(jnp.float32) for s in _SPECIES],
                   axis=1)

    q = 1.0 + _ETA2 * ir * ir
    lnq = jnp.log(q)
    mu = lnr - 0.5 * lnq
    inv_2s2 = 0.5 / lnq
    inv_sig = jax.lax.rsqrt(lnq)
    scale2 = fc * jnp.exp(-_TWO_BODY_DECAY * lnr)
    lnRs2 = np.log(_RS2)
    coef2 = (1.0 / (np.sqrt(2.0 * np.pi) * _RS2)).astype(np.float32)
    pref2 = inv_sig * scale2
    rad2 = jnp.stack(
        [pref2 * float(coef2[s]) *
         jnp.exp(-(float(lnRs2[s]) - mu) ** 2 * inv_2s2)
         for s in range(_NRS2)], axis=-1)
    rep2_blocks = [jnp.sum(rad2 * oh[None, :, t, None], axis=1) for t in range(_NSP)]
    rep2 = jnp.concatenate(rep2_blocks, axis=-1)


    def body(j, accs):
        acc0, acc1 = accs
        r_j = r[j]
        ir_j = ir[j]
        fc_j = fc[j]
        pd_j = pd[j]
        oh_j = oh[j]

        rij = r_j[:, None]
        rjk = r_j[None, :]
        rik = r
        rij2 = rij * rij
        rjk2 = rjk * rjk
        rik2 = rik * rik
        cos_i = jnp.clip((rij2 + rik2 - rjk2) * 0.5 * ir_j[:, None] * ir, -1.0, 1.0)
        cos_j = jnp.clip((rij2 + rjk2 - rik2) * 0.5 * ir_j[:, None] * ir_j[None, :], -1.0, 1.0)
        cos_k = jnp.clip((rik2 + rjk2 - rij2) * 0.5 * ir * ir_j[None, :], -1.0, 1.0)
        sin_i = jnp.sqrt(jnp.clip(1.0 - cos_i * cos_i, 0.0, 1.0))
        atm = (1.0 + 3.0 * cos_i * cos_j * cos_k) * pd_j[:, None] * pd * pd_j[None, :]
        kidx2 = jax.lax.broadcasted_iota(jnp.int32, (A, A), 1)
        neqf = jnp.where(kidx2 != j, 1.0, 0.0)
        pref = atm * fc_j[:, None] * fc * _W3 * neqf
        rmean = 0.5 * (rij + rik)
        rad3 = jnp.stack(
            [jnp.exp(-_ETA3 * (rmean - float(_RS3[s])) ** 2)
             for s in range(_NRS3)], axis=-1)
        a0 = pref * cos_i
        a1 = pref * sin_i
        v0 = a0[..., None] * rad3
        v1 = a1[..., None] * rad3
        c0 = jnp.stack([jnp.sum(v0 * oh[None, :, t, None], axis=1) for t in range(_NSP)], axis=1)
        c1 = jnp.stack([jnp.sum(v1 * oh[None, :, t, None], axis=1) for t in range(_NSP)], axis=1)
        acc0 = acc0 + oh_j[None, :, None, None] * c0[:, None, :, :]
        acc1 = acc1 + oh_j[None, :, None, None] * c1[:, None, :, :]
        return acc0, acc1

    acc0 = jnp.zeros((A, _NSP, _NSP, _NRS3), jnp.float32)
    acc1 = jnp.zeros((A, _NSP, _NSP, _NRS3), jnp.float32)
    for j in range(A):
        acc0, acc1 = body(j, (acc0, acc1))

    blocks = []
    for (s, t) in _PAIRS:
        if s == t:
            b0 = 0.5 * acc0[:, s, s, :]
            b1 = 0.5 * acc1[:, s, s, :]
        else:
            b0 = 0.5 * (acc0[:, s, t, :] + acc0[:, t, s, :])
            b1 = 0.5 * (acc1[:, s, t, :] + acc1[:, t, s, :])
        blocks.append(jnp.stack([b0, b1], axis=-1).reshape(A, _NRS3 * 2))
    rep3 = jnp.concatenate(blocks, axis=-1)

    out_ref[0] = jnp.concatenate([rep2, rep3], axis=-1)


def kernel(X, Z, atomIDs, molIDs, atom_counts):
    Zr = Z.reshape(_NMOL, 1, _MAXA)
    out = pl.pallas_call(
        _mol_kernel,
        grid=(_NMOL,),
        in_specs=[
            pl.BlockSpec((1, _MAXA, 3), lambda m: (m, 0, 0)),
            pl.BlockSpec((1, 1, _MAXA), lambda m: (m, 0, 0)),
        ],
        out_specs=pl.BlockSpec((1, _MAXA, _FP), lambda m: (m, 0, 0)),
        out_shape=jax.ShapeDtypeStruct((_NMOL, _MAXA, _FP), jnp.float32),
    )(X, Zr)
    return out.reshape(_NMOL * _MAXA, _FP)


# speedup vs baseline: 10.2838x; 10.2838x over previous
import numpy as np
import jax
import jax.numpy as jnp
from jax.experimental import pallas as pl

_NMOL = 64
_MAXA = 32
_NSP = 4
_NRS2 = 24
_NRS3 = 20
_RCUT = 8.0
_ETA2 = 0.32
_ETA3 = 2.7
_TWO_BODY_DECAY = 1.8
_THREE_BODY_DECAY = 0.57
_W3 = float(np.sqrt(_ETA3 / np.pi) * 13.4)
_RS2 = np.linspace(0.0, _RCUT, _NRS2 + 1)[1:].astype(np.float32)
_RS3 = np.linspace(0.0, _RCUT, _NRS3 + 1)[1:].astype(np.float32)
_SPECIES = np.array([1.0, 6.0, 7.0, 8.0], dtype=np.float32)
_PAIRS = [(0, 0), (0, 1), (0, 2), (0, 3), (1, 1), (1, 2), (1, 3),
          (2, 2), (2, 3), (3, 3)]
_NPAIR = len(_PAIRS)
_FP = _NSP * _NRS2 + _NPAIR * _NRS3 * 2

_GM = 4
_NG = _NMOL // _GM
_L = _GM * _MAXA

_FOLDNP = np.zeros((_GM * _NSP * _NSP, _GM * _NPAIR), np.float32)
for _b in range(_GM):
    for _p, (_s, _t) in enumerate(_PAIRS):
        _FOLDNP[_b * 16 + _s * 4 + _t, _b * _NPAIR + _p] = 0.5
        _FOLDNP[_b * 16 + _t * 4 + _s, _b * _NPAIR + _p] = 0.5


def _group_kernel(xi_ref, xr_ref, w_ref, ej_ref, fold_ref, o2_ref, o3_ref):
    A = _MAXA
    L = _L
    W = w_ref[0]
    fold = fold_ref[...]

    sub_i = jax.lax.broadcasted_iota(jnp.int32, (A, L), 0)
    lane_c = jax.lax.broadcasted_iota(jnp.int32, (A, L), 1)
    lane_k = jnp.bitwise_and(lane_c, 31)
    eye = sub_i == lane_k

    d2 = jnp.zeros((A, L), jnp.float32)
    for c in range(3):
        xi = xi_ref[0, c]
        xk = xr_ref[0, c][None, :]
        dx = xi - xk
        d2 = d2 + dx * dx
    d2m = jnp.where(eye, 1.0, d2) + 1e-12
    irr = jax.lax.rsqrt(d2m)
    r = d2m * irr
    in_cut = jnp.logical_and(jnp.logical_not(eye), r < _RCUT)
    fc = jnp.where(in_cut, 0.5 * (jnp.cos((np.pi / _RCUT) * r) + 1.0), 0.0)
    lnr = jnp.log(r)
    pd = jnp.exp(-_THREE_BODY_DECAY * lnr)

    q = 1.0 + _ETA2 * irr * irr
    lnq = jnp.log(q)
    mu = lnr - 0.5 * lnq
    inv_2s2 = 0.5 / lnq
    inv_sig = jax.lax.rsqrt(lnq)
    pref2 = inv_sig * fc * jnp.exp(-_TWO_BODY_DECAY * lnr)
    lnRs2 = np.log(_RS2)
    coef2 = (1.0 / (np.sqrt(2.0 * np.pi) * _RS2)).astype(np.float32)
    y2 = jnp.concatenate(
        [pref2 * float(coef2[s]) *
         jnp.exp(-(float(lnRs2[s]) - mu) ** 2 * inv_2s2)
         for s in range(_NRS2)], axis=0)
    o2_ref[0] = jnp.dot(y2, W, preferred_element_type=jnp.float32, precision=jax.lax.Precision.HIGHEST)

    qs = jnp.concatenate([r, irr, fc, pd], axis=0)
    acc = jnp.zeros((2 * _NRS3 * A, _GM * _NSP * _NSP), jnp.float32)
    for j in range(A):
        mj = jnp.where(
            (jax.lax.broadcasted_iota(jnp.int32, (L, L), 0)
             == jnp.bitwise_and(jax.lax.broadcasted_iota(jnp.int32, (L, L), 1),
                                ~jnp.int32(31)) + j),
            1.0, 0.0)
        cb = jnp.dot(qs, mj, preferred_element_type=jnp.float32, precision=jax.lax.Precision.HIGHEST)
        rij = cb[0 * A:1 * A]
        ir_ij = cb[1 * A:2 * A]
        fc_ij = cb[2 * A:3 * A]
        pd_ij = cb[3 * A:4 * A]
        rjk = qs[0 * A + j][None, :]
        ir_jk = qs[1 * A + j][None, :]
        pd_jk = qs[3 * A + j][None, :]

        rij2 = rij * rij
        rjk2 = rjk * rjk
        rik2 = r * r
        cos_i = jnp.clip((rij2 + rik2 - rjk2) * 0.5 * ir_ij * irr, -1.0, 1.0)
        cos_j = jnp.clip((rij2 + rjk2 - rik2) * 0.5 * ir_ij * ir_jk, -1.0, 1.0)
        cos_k = jnp.clip((rik2 + rjk2 - rij2) * 0.5 * irr * ir_jk, -1.0, 1.0)
        sin_i = jnp.sqrt(jnp.clip(1.0 - cos_i * cos_i, 0.0, 1.0))
        atm = (1.0 + 3.0 * cos_i * cos_j * cos_k) * pd_ij * pd * pd_jk
        neq = jnp.where(lane_k == j, 0.0, 1.0)
        pref = atm * fc_ij * fc * (_W3 * neq)
        rmean = 0.5 * (rij + r)
        a0 = pref * cos_i
        a1 = pref * sin_i

        pieces = []
        for s in range(_NRS3):
            rad = jnp.exp(-_ETA3 * (rmean - float(_RS3[s])) ** 2)
            pieces.append(a0 * rad)
        for s in range(_NRS3):
            rad = jnp.exp(-_ETA3 * (rmean - float(_RS3[s])) ** 2)
            pieces.append(a1 * rad)
        ys = jnp.concatenate(pieces, axis=0)
        c = jnp.dot(ys, W, preferred_element_type=jnp.float32, precision=jax.lax.Precision.HIGHEST)
        acc = acc + jnp.dot(c, ej_ref[0, j], preferred_element_type=jnp.float32, precision=jax.lax.Precision.HIGHEST)

    o3_ref[0] = jnp.dot(acc, fold, preferred_element_type=jnp.float32, precision=jax.lax.Precision.HIGHEST)


def kernel(X, Z, atomIDs, molIDs, atom_counts):
    Xg = X.reshape(_NG, _GM, _MAXA, 3)
    XI = jnp.broadcast_to(
        Xg.transpose(0, 3, 2, 1)[:, :, :, :, None],
        (_NG, 3, _MAXA, _GM, _MAXA)).reshape(_NG, 3, _MAXA, _L)
    XR = Xg.transpose(0, 3, 1, 2).reshape(_NG, 3, _L)

    oh = (Z[..., None] == jnp.asarray(_SPECIES)).astype(jnp.float32)
    ohg = oh.reshape(_NG, _GM, _MAXA, _NSP)
    beye = jnp.eye(_GM, dtype=jnp.float32)
    Wm = ohg[:, :, :, None, :] * beye[None, :, None, :, None]
    W = Wm.reshape(_NG, _L, _GM * _NSP)
    teye = jnp.eye(_NSP, dtype=jnp.float32)
    ej = jnp.einsum('bc,tu,gbjs->gjbtcsu', beye, teye, ohg)
    EJ = ej.reshape(_NG, _MAXA, _GM * _NSP, _GM * _NSP * _NSP)

    fold = jnp.asarray(_FOLDNP)

    o2, o3 = pl.pallas_call(
        _group_kernel,
        grid=(_NG,),
        in_specs=[
            pl.BlockSpec((1, 3, _MAXA, _L), lambda g: (g, 0, 0, 0)),
            pl.BlockSpec((1, 3, _L), lambda g: (g, 0, 0)),
            pl.BlockSpec((1, _L, _GM * _NSP), lambda g: (g, 0, 0)),
            pl.BlockSpec((1, _MAXA, _GM * _NSP, _GM * _NSP * _NSP),
                         lambda g: (g, 0, 0, 0)),
            pl.BlockSpec((_GM * _NSP * _NSP, _GM * _NPAIR), lambda g: (0, 0)),
        ],
        out_specs=[
            pl.BlockSpec((1, _NRS2 * _MAXA, _GM * _NSP), lambda g: (g, 0, 0)),
            pl.BlockSpec((1, 2 * _NRS3 * _MAXA, _GM * _NPAIR),
                         lambda g: (g, 0, 0)),
        ],
        out_shape=[
            jax.ShapeDtypeStruct((_NG, _NRS2 * _MAXA, _GM * _NSP), jnp.float32),
            jax.ShapeDtypeStruct((_NG, 2 * _NRS3 * _MAXA, _GM * _NPAIR),
                                 jnp.float32),
        ],
    )(XI, XR, W, EJ, fold)

    rep2 = o2.reshape(_NG, _NRS2, _MAXA, _GM, _NSP) \
             .transpose(0, 3, 2, 4, 1).reshape(_NMOL, _MAXA, _NSP * _NRS2)
    rep3 = o3.reshape(_NG, 2, _NRS3, _MAXA, _GM, _NPAIR) \
             .transpose(0, 4, 3, 5, 2, 1).reshape(_NMOL, _MAXA,
                                                  _NPAIR * _NRS3 * 2)
    out = jnp.concatenate([rep2, rep3], axis=-1)
    return out.reshape(_NMOL * _MAXA, _FP)
